# baseline (speedup 1.0000x reference)
import functools

import jax
import jax.numpy as jnp
from jax import lax
from jax.experimental import pallas as pl
from jax.experimental.pallas import tpu as pltpu
from jax.experimental.pallas import tpu_sc as plsc

_NUM_CORES = 2
_NUM_SUBCORES = 16
_NW = _NUM_CORES * _NUM_SUBCORES
_L = 16

_C = 2048
_UNROLL = 8


def _make_sc_gather(n_rows: int, n_cols: int):
    assert n_rows == 9
    chunk = -(-n_cols // _NW)
    chunk = -(-chunk // 128) * 128
    assert (n_cols - chunk) % 128 == 0
    nb = -(-chunk // _C)
    nb += nb % 2
    last = chunk - _C
    assert chunk <= n_cols and last % 128 == 0 and nb >= 4

    mesh = plsc.VectorSubcoreMesh(
        core_axis_name="c", subcore_axis_name="s", num_cores=_NUM_CORES
    )

    @functools.partial(
        pl.kernel,
        out_type=jax.ShapeDtypeStruct((n_rows, n_cols), jnp.float32),
        mesh=mesh,
        scratch_types=[
            pltpu.VMEM((_L,), jnp.float32),
            [pltpu.VMEM((8, _C), jnp.int32)] * 2,
            [pltpu.VMEM((1, _C), jnp.int32)] * 2,
            [pltpu.VMEM((8, _C), jnp.float32)] * 2,
            [pltpu.VMEM((1, _C), jnp.float32)] * 2,
            [pltpu.SemaphoreType.DMA] * 2,
            [pltpu.SemaphoreType.DMA] * 2,
        ],
        compiler_params=pltpu.CompilerParams(needs_layout_passes=False),
    )
    def sc_gather(w_hbm, idx_hbm, out_hbm, wv, ib8, ib1, ob8, ob1, isem, osem):
        wid = lax.axis_index("s") * _NUM_CORES + lax.axis_index("c")
        base = jnp.minimum(wid * chunk, n_cols - chunk)

        def c0(b):
            return base + jnp.minimum(b * _C, last)

        def in_at(b):
            c = c0(b)
            return (
                idx_hbm.at[pl.ds(0, 8), pl.ds(c, _C)],
                idx_hbm.at[pl.ds(8, 1), pl.ds(c, _C)],
            )

        def out_at(b):
            c = c0(b)
            return (
                out_hbm.at[pl.ds(0, 8), pl.ds(c, _C)],
                out_hbm.at[pl.ds(8, 1), pl.ds(c, _C)],
            )

        def start_in(b, p):
            s8, s1 = in_at(b)
            pltpu.async_copy(s8, ib8[p], isem[p])
            pltpu.async_copy(s1, ib1[p], isem[p])

        def wait_in(b, p):
            s8, s1 = in_at(b)
            pltpu.make_async_copy(s8, ib8[p], isem[p]).wait()
            pltpu.make_async_copy(s1, ib1[p], isem[p]).wait()

        def start_out(b, p):
            d8, d1 = out_at(b)
            pltpu.async_copy(ob8[p], d8, osem[p])
            pltpu.async_copy(ob1[p], d1, osem[p])

        def wait_out(b, p):
            d8, d1 = out_at(b)
            pltpu.make_async_copy(ob8[p], d8, osem[p]).wait()
            pltpu.make_async_copy(ob1[p], d1, osem[p]).wait()

        start_in(0, 0)
        start_in(1, 1)
        pltpu.sync_copy(w_hbm, wv.at[pl.ds(0, 9)])

        def gather_block(p):
            @plsc.parallel_loop(0, _C, _L, unroll=4)
            def _(o):
                for r in range(8):
                    idx = ib8[p][r, pl.ds(o, _L)]
                    ob8[p][r, pl.ds(o, _L)] = plsc.load_gather(wv, [idx])
                idx1 = ib1[p][0, pl.ds(o, _L)]
                ob1[p][0, pl.ds(o, _L)] = plsc.load_gather(wv, [idx1])

        for p in range(2):
            wait_in(p, p)
            gather_block(p)
            start_out(p, p)
            start_in(p + 2, p)

        def bb_body(bb, _):
            b = bb * 2
            for p in range(2):
                wait_out(b + p - 2, p)
                wait_in(b + p, p)
                gather_block(p)
                start_out(b + p, p)
                start_in(b + p + 2, p)
            return _

        lax.fori_loop(1, nb // 2, bb_body, None, unroll=False)

        for p in range(2):
            wait_in(nb + p, p)
            wait_out(nb - 2 + p, p)

    return sc_gather


def kernel(weight, indices):
    it = indices.T
    out_t = _make_sc_gather(*it.shape)(weight, it)
    return out_t.T


# speedup vs baseline: 1.0757x; 1.0757x over previous
import functools

import jax
import jax.numpy as jnp
from jax import lax
from jax.experimental import pallas as pl
from jax.experimental.pallas import tpu as pltpu
from jax.experimental.pallas import tpu_sc as plsc

_NUM_CORES = 2
_NUM_SUBCORES = 16
_NW = _NUM_CORES * _NUM_SUBCORES
_L = 16

_C = 2048
_UNROLL = 8


def _make_sc_gather(n_rows: int, n_cols: int):
    assert n_rows == 9
    chunk = -(-n_cols // _NW)
    chunk = -(-chunk // 128) * 128
    assert (n_cols - chunk) % 128 == 0
    nb = -(-chunk // _C)
    nb += nb % 2
    last = chunk - _C
    assert chunk <= n_cols and last % 128 == 0 and nb >= 4

    mesh = plsc.VectorSubcoreMesh(
        core_axis_name="c", subcore_axis_name="s", num_cores=_NUM_CORES
    )

    @functools.partial(
        pl.kernel,
        out_type=jax.ShapeDtypeStruct((n_rows, n_cols), jnp.float32),
        mesh=mesh,
        scratch_types=[
            pltpu.VMEM((_L,), jnp.float32),
            [pltpu.VMEM((8, _C), jnp.int32)] * 2,
            [pltpu.VMEM((1, _C), jnp.int32)] * 2,
            [pltpu.VMEM((8, _C), jnp.float32)] * 2,
            [pltpu.VMEM((1, _C), jnp.float32)] * 2,
            [pltpu.SemaphoreType.DMA] * 2,
            [pltpu.SemaphoreType.DMA] * 2,
        ],
        compiler_params=pltpu.CompilerParams(needs_layout_passes=False),
    )
    def sc_gather(w_hbm, idx_hbm, out_hbm, wv, ib8, ib1, ob8, ob1, isem, osem):
        wid = lax.axis_index("s") * _NUM_CORES + lax.axis_index("c")
        base = jnp.minimum(wid * chunk, n_cols - chunk)

        def c0(b):
            return base + jnp.minimum(b * _C, last)

        def in_at(b):
            c = c0(b)
            return (
                idx_hbm.at[pl.ds(0, 8), pl.ds(c, _C)],
                idx_hbm.at[pl.ds(8, 1), pl.ds(c, _C)],
            )

        def out_at(b):
            c = c0(b)
            return (
                out_hbm.at[pl.ds(0, 8), pl.ds(c, _C)],
                out_hbm.at[pl.ds(8, 1), pl.ds(c, _C)],
            )

        def start_in(b, p):
            s8, s1 = in_at(b)
            pltpu.async_copy(s8, ib8[p], isem[p])
            pltpu.async_copy(s1, ib1[p], isem[p])

        def wait_in(b, p):
            s8, s1 = in_at(b)
            pltpu.make_async_copy(s8, ib8[p], isem[p]).wait()
            pltpu.make_async_copy(s1, ib1[p], isem[p]).wait()

        def start_out(b, p):
            d8, d1 = out_at(b)
            pltpu.async_copy(ob8[p], d8, osem[p])
            pltpu.async_copy(ob1[p], d1, osem[p])

        def wait_out(b, p):
            d8, d1 = out_at(b)
            pltpu.make_async_copy(ob8[p], d8, osem[p]).wait()
            pltpu.make_async_copy(ob1[p], d1, osem[p]).wait()

        start_in(0, 0)
        start_in(1, 1)
        pltpu.sync_copy(w_hbm, wv.at[pl.ds(0, 9)])

        def gather_block(p):
            @plsc.parallel_loop(0, _C, _L, unroll=1)
            def _(o):
                for r in range(8):
                    idx = ib8[p][r, pl.ds(o, _L)]
                    ob8[p][r, pl.ds(o, _L)] = plsc.load_gather(wv, [idx])
                idx1 = ib1[p][0, pl.ds(o, _L)]
                ob1[p][0, pl.ds(o, _L)] = plsc.load_gather(wv, [idx1])

        for p in range(2):
            wait_in(p, p)
            gather_block(p)
            start_out(p, p)
            start_in(p + 2, p)

        def bb_body(bb, _):
            b = bb * 2
            for p in range(2):
                wait_out(b + p - 2, p)
                wait_in(b + p, p)
                gather_block(p)
                start_out(b + p, p)
                start_in(b + p + 2, p)
            return _

        lax.fori_loop(1, nb // 2, bb_body, None, unroll=False)

        for p in range(2):
            wait_in(nb + p, p)
            wait_out(nb - 2 + p, p)

    return sc_gather


def kernel(weight, indices):
    it = indices.T
    out_t = _make_sc_gather(*it.shape)(weight, it)
    return out_t.T


# speedup vs baseline: 1.0886x; 1.0120x over previous
import functools

import jax
import jax.numpy as jnp
from jax import lax
from jax.experimental import pallas as pl
from jax.experimental.pallas import tpu as pltpu
from jax.experimental.pallas import tpu_sc as plsc

_NUM_CORES = 2
_NUM_SUBCORES = 16
_NW = _NUM_CORES * _NUM_SUBCORES
_L = 16

_C = 2048
_UNROLL = 8


def _make_sc_gather(n_rows: int, n_cols: int):
    assert n_rows == 9
    chunk = -(-n_cols // _NW)
    chunk = -(-chunk // 128) * 128
    assert (n_cols - chunk) % 128 == 0
    nb = -(-chunk // _C)
    nb += nb % 2
    last = chunk - _C
    assert chunk <= n_cols and last % 128 == 0 and nb >= 4

    mesh = plsc.VectorSubcoreMesh(
        core_axis_name="c", subcore_axis_name="s", num_cores=_NUM_CORES
    )

    @functools.partial(
        pl.kernel,
        out_type=jax.ShapeDtypeStruct((n_rows, n_cols), jnp.float32),
        mesh=mesh,
        scratch_types=[
            pltpu.VMEM((_L,), jnp.float32),
            [pltpu.VMEM((8, _C), jnp.int32)] * 2,
            [pltpu.VMEM((1, _C), jnp.int32)] * 2,
            [pltpu.VMEM((8, _C), jnp.float32)] * 2,
            [pltpu.VMEM((1, _C), jnp.float32)] * 2,
            [pltpu.SemaphoreType.DMA] * 2,
            [pltpu.SemaphoreType.DMA] * 2,
        ],
        compiler_params=pltpu.CompilerParams(needs_layout_passes=False),
    )
    def sc_gather(w_hbm, idx_hbm, out_hbm, wv, ib8, ib1, ob8, ob1, isem, osem):
        wid = lax.axis_index("s") * _NUM_CORES + lax.axis_index("c")
        base = jnp.minimum(wid * chunk, n_cols - chunk)

        def c0(b):
            return base + jnp.minimum(b * _C, last)

        def in_at(b):
            c = c0(b)
            return (
                idx_hbm.at[pl.ds(0, 8), pl.ds(c, _C)],
                idx_hbm.at[pl.ds(8, 1), pl.ds(c, _C)],
            )

        def out_at(b):
            c = c0(b)
            return (
                out_hbm.at[pl.ds(0, 8), pl.ds(c, _C)],
                out_hbm.at[pl.ds(8, 1), pl.ds(c, _C)],
            )

        def start_in(b, p):
            s8, s1 = in_at(b)
            pltpu.async_copy(s8, ib8[p], isem[p])
            pltpu.async_copy(s1, ib1[p], isem[p])

        def wait_in(b, p):
            s8, s1 = in_at(b)
            pltpu.make_async_copy(s8, ib8[p], isem[p]).wait()
            pltpu.make_async_copy(s1, ib1[p], isem[p]).wait()

        def start_out(b, p):
            d8, d1 = out_at(b)
            pltpu.async_copy(ob8[p], d8, osem[p])
            pltpu.async_copy(ob1[p], d1, osem[p])

        def wait_out(b, p):
            d8, d1 = out_at(b)
            pltpu.make_async_copy(ob8[p], d8, osem[p]).wait()
            pltpu.make_async_copy(ob1[p], d1, osem[p]).wait()

        start_in(0, 0)
        start_in(1, 1)
        pltpu.sync_copy(w_hbm, wv.at[pl.ds(0, 9)])

        def gather_block(p):
            @plsc.parallel_loop(0, _C, _L, unroll=1)
            def _(o):
                for r in range(8):
                    idx = ib8[p][r, pl.ds(o, _L)]
                    ob8[p][r, pl.ds(o, _L)] = plsc.load_gather(wv, [idx])
                idx1 = ib1[p][0, pl.ds(o, _L)]
                ob1[p][0, pl.ds(o, _L)] = plsc.load_gather(wv, [idx1])

        def bb_body(bb, _):
            b = bb * 2
            for p in range(2):
                @pl.when(bb > 0)
                def _():
                    wait_out(b + p - 2, p)

                wait_in(b + p, p)
                gather_block(p)
                start_out(b + p, p)
                start_in(b + p + 2, p)
            return _

        lax.fori_loop(0, nb // 2, bb_body, None, unroll=False)

        for p in range(2):
            wait_in(nb + p, p)
            wait_out(nb - 2 + p, p)

    return sc_gather


def kernel(weight, indices):
    it = indices.T
    out_t = _make_sc_gather(*it.shape)(weight, it)
    return out_t.T


# speedup vs baseline: 1.1133x; 1.0227x over previous
import functools

import jax
import jax.numpy as jnp
from jax import lax
from jax.experimental import pallas as pl
from jax.experimental.pallas import tpu as pltpu
from jax.experimental.pallas import tpu_sc as plsc

_NUM_CORES = 2
_NUM_SUBCORES = 16
_NW = _NUM_CORES * _NUM_SUBCORES
_L = 16

_C = 2048
_UNROLL = 8


def _make_sc_gather(n_rows: int, n_cols: int):
    assert n_rows == 9
    chunk = -(-n_cols // _NW)
    chunk = -(-chunk // 128) * 128
    assert (n_cols - chunk) % 128 == 0
    nb = -(-chunk // _C)
    nb += nb % 2
    last = chunk - _C
    assert chunk <= n_cols and last % 128 == 0 and nb >= 4

    mesh = plsc.VectorSubcoreMesh(
        core_axis_name="c", subcore_axis_name="s", num_cores=_NUM_CORES
    )

    @functools.partial(
        pl.kernel,
        out_type=jax.ShapeDtypeStruct((n_rows, n_cols), jnp.float32),
        mesh=mesh,
        scratch_types=[
            pltpu.VMEM((_L,), jnp.float32),
            [pltpu.VMEM((8, _C), jnp.int32)] * 2,
            [pltpu.VMEM((1, _C), jnp.int32)] * 2,
            [pltpu.VMEM((8, _C), jnp.float32)] * 2,
            [pltpu.VMEM((1, _C), jnp.float32)] * 2,
            [pltpu.SemaphoreType.DMA] * 2,
            [pltpu.SemaphoreType.DMA] * 2,
        ],
        compiler_params=pltpu.CompilerParams(needs_layout_passes=False),
    )
    def sc_gather(w_hbm, idx_hbm, out_hbm, wv, ib8, ib1, ob8, ob1, isem, osem):
        wid = lax.axis_index("s") * _NUM_CORES + lax.axis_index("c")
        base = jnp.minimum(wid * chunk, n_cols - chunk)

        def c0(b):
            return base + jnp.minimum(b * _C, last)

        def in_at(b):
            c = c0(b)
            return (
                idx_hbm.at[pl.ds(0, 8), pl.ds(c, _C)],
                idx_hbm.at[pl.ds(8, 1), pl.ds(c, _C)],
            )

        def out_at(b):
            c = c0(b)
            return (
                out_hbm.at[pl.ds(0, 8), pl.ds(c, _C)],
                out_hbm.at[pl.ds(8, 1), pl.ds(c, _C)],
            )

        def start_in(b, p):
            s8, s1 = in_at(b)
            pltpu.async_copy(s8, ib8[p], isem[p])
            pltpu.async_copy(s1, ib1[p], isem[p])

        def wait_in(b, p):
            s8, s1 = in_at(b)
            pltpu.make_async_copy(s8, ib8[p], isem[p]).wait()
            pltpu.make_async_copy(s1, ib1[p], isem[p]).wait()

        def start_out(b, p):
            d8, d1 = out_at(b)
            pltpu.async_copy(ob8[p], d8, osem[p])
            pltpu.async_copy(ob1[p], d1, osem[p])

        def wait_out(b, p):
            d8, d1 = out_at(b)
            pltpu.make_async_copy(ob8[p], d8, osem[p]).wait()
            pltpu.make_async_copy(ob1[p], d1, osem[p]).wait()

        start_in(0, 0)
        start_in(1, 1)
        pltpu.sync_copy(w_hbm, wv.at[pl.ds(0, 9)])

        def gather_block(p):
            @plsc.parallel_loop(0, _C, _L, unroll=1)
            def _(o):
                for r in range(8):
                    idx = ib8[p][r, pl.ds(o, _L)]
                    ob8[p][r, pl.ds(o, _L)] = plsc.load_gather(wv, [idx])
                idx1 = ib1[p][0, pl.ds(o, _L)]
                ob1[p][0, pl.ds(o, _L)] = plsc.load_gather(wv, [idx1])

        def bb_body(bb, _):
            b = bb * 2
            for p in range(2):
                @pl.when(bb > 0)
                def _():
                    wait_out(b + p - 2, p)

                wait_in(b + p, p)
                gather_block(p)
                start_out(b + p, p)

                @pl.when(bb < nb // 2 - 1)
                def _():
                    start_in(b + p + 2, p)

            return _

        lax.fori_loop(0, nb // 2, bb_body, None, unroll=False)

        for p in range(2):
            wait_out(nb - 2 + p, p)

    return sc_gather


def kernel(weight, indices):
    it = indices.T
    out_t = _make_sc_gather(*it.shape)(weight, it)
    return out_t.T
